# baseline (speedup 1.0000x reference)
import functools

import jax
import jax.numpy as jnp
from jax import lax
from jax.experimental import pallas as pl
from jax.experimental.pallas import tpu as pltpu
from jax.experimental.pallas import tpu_sc as plsc

NUM_DATASETS = 26
EMB = 128
BATCH = 16384

B_SC = 8192
B_TC = BATCH - B_SC

_info = plsc.get_sparse_core_info()
_NC, _NS = _info.num_cores, _info.num_subcores
_NW = _NC * _NS
_B_PER_W = B_SC // _NW
_S = 64
_C = _B_PER_W // _S

_mesh = plsc.VectorSubcoreMesh(core_axis_name="c", subcore_axis_name="s")


@functools.partial(
    pl.kernel,
    mesh=_mesh,
    out_type=jax.ShapeDtypeStruct((B_SC, EMB), jnp.float32),
    scratch_types=[
        pltpu.VMEM((_C, _S), jnp.int32),
        pltpu.VMEM((2, _S, EMB), jnp.float32),
        pltpu.VMEM_SHARED((NUM_DATASETS, EMB), jnp.float32),
        pltpu.SemaphoreType.DMA,
        pltpu.SemaphoreType.DMA,
        pltpu.SemaphoreType.DMA,
    ],
)
def _sc_gather(idx_hbm, table_hbm, out_hbm, idx_v, buf, table_sh, gsem,
               wsem0, wsem1):
    sid = lax.axis_index("s")
    wid = sid * _NC + lax.axis_index("c")
    base = wid * _B_PER_W

    @pl.when(sid == 0)
    def _():
        pltpu.sync_copy(table_hbm, table_sh)

    pltpu.sync_copy(idx_hbm.at[wid], idx_v)
    plsc.subcore_barrier()

    wsems = (wsem0, wsem1)
    writes = [None, None]
    for k in range(_C):
        b = k % 2
        if writes[b] is not None:
            writes[b].wait()
        pltpu.async_copy(table_sh.at[idx_v.at[k]], buf.at[b], gsem).wait()
        writes[b] = pltpu.async_copy(
            buf.at[b], out_hbm.at[pl.ds(base + k * _S, _S)], wsems[b])
    writes[(_C - 1) % 2].wait()
    writes[_C % 2].wait()


_PAD = 32
_BLK = 2048


def _tc_body(idx_ref, table_ref, out_ref):
    idx = idx_ref[...]
    one_hot = (idx == lax.broadcasted_iota(jnp.int32, (_BLK, _PAD), 1))
    out_ref[...] = jnp.dot(one_hot.astype(jnp.float32), table_ref[...],
                           preferred_element_type=jnp.float32)


_tc_lookup = pl.pallas_call(
    _tc_body,
    grid=(B_TC // _BLK,),
    in_specs=[
        pl.BlockSpec((_BLK, 1), lambda i: (i, 0)),
        pl.BlockSpec((_PAD, EMB), lambda i: (0, 0)),
    ],
    out_specs=pl.BlockSpec((_BLK, EMB), lambda i: (i, 0)),
    out_shape=jax.ShapeDtypeStruct((B_TC, EMB), jnp.float32),
)


def kernel(dataset_indices, table):
    idx = dataset_indices.astype(jnp.int32)
    table_p = jnp.pad(table, ((0, _PAD - NUM_DATASETS), (0, 0)))
    sc_out = _sc_gather(idx[:B_SC].reshape(_NW, _C, _S), table)
    tc_out = _tc_lookup(idx[B_SC:].reshape(B_TC, 1), table_p)
    return jnp.concatenate([sc_out, tc_out], axis=0)


# speedup vs baseline: 1.3035x; 1.3035x over previous
import functools

import jax
import jax.numpy as jnp
from jax import lax
from jax.experimental import pallas as pl
from jax.experimental.pallas import tpu as pltpu
from jax.experimental.pallas import tpu_sc as plsc

NUM_DATASETS = 26
EMB = 128
BATCH = 16384

_info = plsc.get_sparse_core_info()
_NC, _NS = _info.num_cores, _info.num_subcores
_NW = _NC * _NS
_B_PER_W = BATCH // _NW
_S = 64
_C = _B_PER_W // _S
_REP = 32

_mesh = plsc.VectorSubcoreMesh(core_axis_name="c", subcore_axis_name="s")


@functools.partial(
    pl.kernel,
    mesh=_mesh,
    out_type=jax.ShapeDtypeStruct((BATCH, EMB), jnp.float32),
    scratch_types=[
        pltpu.VMEM((_C, _S), jnp.int32),
        pltpu.VMEM((2, _S, EMB), jnp.float32),
        pltpu.VMEM_SHARED((_NS * _REP, EMB), jnp.float32),
        pltpu.SemaphoreType.DMA,
        pltpu.SemaphoreType.DMA,
        pltpu.SemaphoreType.DMA,
    ],
)
def _gather_kernel(idx_hbm, table_hbm, out_hbm, idx_v, buf, table_sh, gsem,
                   wsem0, wsem1):
    sid = lax.axis_index("s")
    wid = sid * _NC + lax.axis_index("c")
    base = wid * _B_PER_W

    pltpu.sync_copy(idx_hbm.at[wid], idx_v)
    pltpu.sync_copy(table_hbm, table_sh.at[pl.ds(sid * _REP, NUM_DATASETS)])

    off = sid * _REP
    for k in range(_C):
        for j in range(_S // 16):
            sl = pl.ds(j * 16, 16)
            idx_v[k, sl] = idx_v[k, sl] + off

    wsems = (wsem0, wsem1)
    writes = [None, None]
    for k in range(_C):
        b = k % 2
        if writes[b] is not None:
            writes[b].wait()
        pltpu.async_copy(table_sh.at[idx_v.at[k]], buf.at[b], gsem).wait()
        writes[b] = pltpu.async_copy(
            buf.at[b], out_hbm.at[pl.ds(base + k * _S, _S)], wsems[b])
    writes[(_C - 1) % 2].wait()
    writes[_C % 2].wait()


def kernel(dataset_indices, table):
    idx = dataset_indices.astype(jnp.int32).reshape(_NW, _C, _S)
    return _gather_kernel(idx, table)


# speedup vs baseline: 1.3473x; 1.0336x over previous
import functools

import jax
import jax.numpy as jnp
from jax import lax
from jax.experimental import pallas as pl
from jax.experimental.pallas import tpu as pltpu
from jax.experimental.pallas import tpu_sc as plsc

NUM_DATASETS = 26
EMB = 128
BATCH = 16384

_info = plsc.get_sparse_core_info()
_NC, _NS = _info.num_cores, _info.num_subcores
_NW = _NC * _NS
_B_PER_W = BATCH // _NW
_S = 64
_C = _B_PER_W // _S

_mesh = plsc.VectorSubcoreMesh(core_axis_name="c", subcore_axis_name="s")


@functools.partial(
    pl.kernel,
    mesh=_mesh,
    out_type=jax.ShapeDtypeStruct((BATCH, EMB), jnp.float32),
    scratch_types=[
        pltpu.VMEM((_C, _S), jnp.int32),
        pltpu.VMEM((2, _S, EMB), jnp.float32),
        pltpu.VMEM_SHARED((NUM_DATASETS, EMB), jnp.float32),
        pltpu.SemaphoreType.DMA,
        pltpu.SemaphoreType.DMA,
        pltpu.SemaphoreType.DMA,
    ],
)
def _gather_kernel(idx_hbm, table_hbm, out_hbm, idx_v, buf, table_sh, gsem,
                   wsem0, wsem1):
    sid = lax.axis_index("s")
    wid = sid * _NC + lax.axis_index("c")
    base = wid * _B_PER_W

    @pl.when(sid == 0)
    def _():
        pltpu.sync_copy(table_hbm, table_sh)

    pltpu.sync_copy(idx_hbm.at[wid], idx_v)
    plsc.subcore_barrier()

    wsems = (wsem0, wsem1)
    writes = [None, None]
    for k in range(_C):
        b = k % 2
        if writes[b] is not None:
            writes[b].wait()
        pltpu.async_copy(table_sh.at[idx_v.at[k]], buf.at[b], gsem).wait()
        writes[b] = pltpu.async_copy(
            buf.at[b], out_hbm.at[pl.ds(base + k * _S, _S)], wsems[b])
    writes[(_C - 1) % 2].wait()
    writes[_C % 2].wait()


def kernel(dataset_indices, table):
    idx = dataset_indices.astype(jnp.int32).reshape(_NW, _C, _S)
    return _gather_kernel(idx, table)
